# Initial kernel scaffold; baseline (speedup 1.0000x reference)
import jax
import jax.numpy as jnp
from jax.experimental import pallas as pl


def kernel(feature_map, rois):
    raise NotImplementedError("write your pallas kernel here")


# speedup vs baseline: 8.8419x; 8.8419x over previous
import jax
import jax.numpy as jnp
import numpy as np
from jax.experimental import pallas as pl
from jax.experimental.pallas import tpu as pltpu

S = 7
SLICE = 16
RB = 8

NEG = float(np.finfo(np.float32).min)


def _roi_kernel(boxes_ref, fm_ref, out_ref, p1_ref):
    b = pl.program_id(0)
    rblk = pl.program_id(1)
    _, W, H, C = fm_ref.shape
    R_total = out_ref.shape[1] * pl.num_programs(1)

    for rr in range(RB):
        base = (b * R_total + rblk * RB + rr) * 4
        x1 = boxes_ref[base + 0]
        y1 = boxes_ref[base + 1]
        x2 = boxes_ref[base + 2]
        y2 = boxes_ref[base + 3]
        nx = x2 - x1 + 1
        ny = y2 - y1 + 1

        for i in range(S):
            sx = x1 + (i * nx) // S
            ex = x1 - ((-(i + 1) * nx) // S)
            s0 = jnp.minimum(sx, W - SLICE)
            sl = fm_ref[0, pl.ds(s0, SLICE)]
            aw = s0 + jax.lax.broadcasted_iota(jnp.int32, (SLICE, 1, 1), 0)
            m = (aw >= sx) & (aw < ex)
            p1_ref[i] = jnp.max(jnp.where(m, sl, NEG), axis=0)

        p1 = p1_ref[...]
        ah = jax.lax.broadcasted_iota(jnp.int32, (1, H, 1), 1)
        for j in range(S):
            sy = y1 + (j * ny) // S
            ey = y1 - ((-(j + 1) * ny) // S)
            m2 = (ah >= sy) & (ah < ey)
            out_ref[0, rr, :, j, :] = jnp.max(jnp.where(m2, p1, NEG), axis=1)


def kernel(feature_map, rois):
    B, C, W, H = feature_map.shape
    R = rois.shape[1]
    fm_t = jnp.transpose(feature_map, (0, 2, 3, 1))
    boxes = rois.astype(jnp.int32).reshape(-1)

    out = pl.pallas_call(
        _roi_kernel,
        grid=(B, R // RB),
        in_specs=[
            pl.BlockSpec(memory_space=pltpu.SMEM),
            pl.BlockSpec((1, W, H, C), lambda b, r: (b, 0, 0, 0)),
        ],
        out_specs=pl.BlockSpec((1, RB, S, S, C), lambda b, r: (b, r, 0, 0, 0)),
        out_shape=jax.ShapeDtypeStruct((B, R, S, S, C), jnp.float32),
        scratch_shapes=[pltpu.VMEM((S, H, C), jnp.float32)],
        compiler_params=pltpu.CompilerParams(
            dimension_semantics=("parallel", "arbitrary"),
        ),
        name="roi_pool",
    )(boxes, fm_t)
    return jnp.transpose(out, (0, 1, 4, 2, 3))


# speedup vs baseline: 19.5291x; 2.2087x over previous
import jax
import jax.numpy as jnp
import numpy as np
from jax.experimental import pallas as pl
from jax.experimental.pallas import tpu as pltpu

S = 7
RB = 8
HWIN = 24

NEG = float(np.finfo(np.float32).min)


def _roi_kernel(boxes_ref, fm_ref, out_ref, p1_ref, tbl_ref):
    b = pl.program_id(0)
    rblk = pl.program_id(1)
    _, W, H, C = fm_ref.shape
    R_total = out_ref.shape[1] * pl.num_programs(1)

    @pl.when(rblk == 0)
    def _build():
        tbl_ref[0] = fm_ref[0]
        tbl_ref[1, 0:63] = jnp.maximum(tbl_ref[0, 0:63], tbl_ref[0, 1:64])
        tbl_ref[2, 0:61] = jnp.maximum(tbl_ref[1, 0:61], tbl_ref[1, 2:63])
        tbl_ref[3, 0:57] = jnp.maximum(tbl_ref[2, 0:57], tbl_ref[2, 4:61])

    for rr in range(RB):
        base = (b * R_total + rblk * RB + rr) * 4
        x1 = boxes_ref[base + 0]
        y1 = boxes_ref[base + 1]
        x2 = boxes_ref[base + 2]
        y2 = boxes_ref[base + 3]
        nx = x2 - x1 + 1
        ny = y2 - y1 + 1

        for i in range(S):
            sx = x1 + (i * nx) // S
            ex = x1 - ((-(i + 1) * nx) // S)
            w = ex - sx
            lvl = ((w >= 2).astype(jnp.int32) + (w >= 4).astype(jnp.int32)
                   + (w >= 8).astype(jnp.int32))
            p = jnp.where(w >= 8, 8, jnp.where(w >= 4, 4, jnp.where(w >= 2, 2, 1)))
            p1_ref[i] = jnp.maximum(tbl_ref[lvl, sx], tbl_ref[lvl, ex - p])

        for j in range(S):
            sy = y1 + (j * ny) // S
            ey = y1 - ((-(j + 1) * ny) // S)
            t8 = pl.multiple_of(jnp.minimum((sy >> 3) << 3, H - HWIN), 8)
            sl2 = p1_ref[:, pl.ds(t8, HWIN), :]
            ah = t8 + jax.lax.broadcasted_iota(jnp.int32, (1, HWIN, 1), 1)
            m2 = (ah >= sy) & (ah < ey)
            out_ref[0, rr, :, j, :] = jnp.max(jnp.where(m2, sl2, NEG), axis=1)


def kernel(feature_map, rois):
    B, C, W, H = feature_map.shape
    R = rois.shape[1]
    fm_t = jnp.transpose(feature_map, (0, 2, 3, 1))
    boxes = rois.astype(jnp.int32).reshape(-1)

    out = pl.pallas_call(
        _roi_kernel,
        grid=(B, R // RB),
        in_specs=[
            pl.BlockSpec(memory_space=pltpu.SMEM),
            pl.BlockSpec((1, W, H, C), lambda b, r: (b, 0, 0, 0)),
        ],
        out_specs=pl.BlockSpec((1, RB, S, S, C), lambda b, r: (b, r, 0, 0, 0)),
        out_shape=jax.ShapeDtypeStruct((B, R, S, S, C), jnp.float32),
        scratch_shapes=[
            pltpu.VMEM((S, H, C), jnp.float32),
            pltpu.VMEM((4, W, H, C), jnp.float32),
        ],
        compiler_params=pltpu.CompilerParams(
            dimension_semantics=("parallel", "arbitrary"),
        ),
        name="roi_pool",
    )(boxes, fm_t)
    return jnp.transpose(out, (0, 1, 4, 2, 3))


# speedup vs baseline: 20.0586x; 1.0271x over previous
import jax
import jax.numpy as jnp
import numpy as np
from jax.experimental import pallas as pl
from jax.experimental.pallas import tpu as pltpu

S = 7
RB = 16
HWIN = 24

NEG = float(np.finfo(np.float32).min)


def _roi_kernel(boxes_ref, fm_ref, out_ref, p1_ref, tbl_ref):
    b = pl.program_id(0)
    rblk = pl.program_id(1)
    _, W, H, C = fm_ref.shape
    R_total = out_ref.shape[1] * pl.num_programs(1)

    @pl.when(rblk == 0)
    def _build():
        tbl_ref[0] = fm_ref[0]
        tbl_ref[1, 0:63] = jnp.maximum(tbl_ref[0, 0:63], tbl_ref[0, 1:64])
        tbl_ref[2, 0:61] = jnp.maximum(tbl_ref[1, 0:61], tbl_ref[1, 2:63])
        tbl_ref[3, 0:57] = jnp.maximum(tbl_ref[2, 0:57], tbl_ref[2, 4:61])

    for rr in range(RB):
        base = (b * R_total + rblk * RB + rr) * 4
        x1 = boxes_ref[base + 0]
        y1 = boxes_ref[base + 1]
        x2 = boxes_ref[base + 2]
        y2 = boxes_ref[base + 3]
        nx = x2 - x1 + 1
        ny = y2 - y1 + 1

        for i in range(S):
            sx = x1 + (i * nx) // S
            ex = x1 - ((-(i + 1) * nx) // S)
            w = ex - sx
            lvl = ((w >= 2).astype(jnp.int32) + (w >= 4).astype(jnp.int32)
                   + (w >= 8).astype(jnp.int32))
            p = jnp.where(w >= 8, 8, jnp.where(w >= 4, 4, jnp.where(w >= 2, 2, 1)))
            p1_ref[i] = jnp.maximum(tbl_ref[lvl, sx], tbl_ref[lvl, ex - p])

        rows = []
        for j in range(S):
            sy = y1 + (j * ny) // S
            ey = y1 - ((-(j + 1) * ny) // S)
            t8 = pl.multiple_of(jnp.minimum((sy >> 3) << 3, H - HWIN), 8)
            sl2 = p1_ref[:, pl.ds(t8, HWIN), :]
            ah = t8 + jax.lax.broadcasted_iota(jnp.int32, (1, HWIN, 1), 1)
            m2 = (ah >= sy) & (ah < ey)
            rows.append(jnp.max(jnp.where(m2, sl2, NEG), axis=1))
        out_ref[0, rr] = jnp.stack(rows, axis=0)


def kernel(feature_map, rois):
    B, C, W, H = feature_map.shape
    R = rois.shape[1]
    fm_t = jnp.transpose(feature_map, (0, 2, 3, 1))
    boxes = rois.astype(jnp.int32).reshape(-1)

    out = pl.pallas_call(
        _roi_kernel,
        grid=(B, R // RB),
        in_specs=[
            pl.BlockSpec(memory_space=pltpu.SMEM),
            pl.BlockSpec((1, W, H, C), lambda b, r: (b, 0, 0, 0)),
        ],
        out_specs=pl.BlockSpec((1, RB, S, S, C), lambda b, r: (b, r, 0, 0, 0)),
        out_shape=jax.ShapeDtypeStruct((B, R, S, S, C), jnp.float32),
        scratch_shapes=[
            pltpu.VMEM((S, H, C), jnp.float32),
            pltpu.VMEM((4, W, H, C), jnp.float32),
        ],
        compiler_params=pltpu.CompilerParams(
            dimension_semantics=("parallel", "arbitrary"),
        ),
        name="roi_pool",
    )(boxes, fm_t)
    return jnp.transpose(out, (0, 1, 4, 3, 2))


# speedup vs baseline: 22.5090x; 1.1222x over previous
import jax
import jax.numpy as jnp
import numpy as np
from jax.experimental import pallas as pl
from jax.experimental.pallas import tpu as pltpu

S = 7
RB = 16
HWIN = 24
MW = 5 * S

NEG = float(np.finfo(np.float32).min)


def _roi_kernel(meta_ref, fm_ref, out_ref, p1_ref, tbl_ref):
    b = pl.program_id(0)
    rblk = pl.program_id(1)
    _, W, H, C = fm_ref.shape
    R_total = out_ref.shape[1] * pl.num_programs(1)

    @pl.when(rblk == 0)
    def _build():
        tbl_ref[0 * W:0 * W + 64] = fm_ref[0]
        tbl_ref[1 * W:1 * W + 63] = jnp.maximum(tbl_ref[0:63], tbl_ref[1:64])
        tbl_ref[2 * W:2 * W + 61] = jnp.maximum(tbl_ref[W:W + 61],
                                                tbl_ref[W + 2:W + 63])
        tbl_ref[3 * W:3 * W + 57] = jnp.maximum(tbl_ref[2 * W:2 * W + 57],
                                                tbl_ref[2 * W + 4:2 * W + 61])

    for rr in range(RB):
        base = (b * R_total + rblk * RB + rr) * MW

        for i in range(S):
            a1 = meta_ref[base + i]
            a2 = meta_ref[base + S + i]
            p1_ref[i] = jnp.maximum(tbl_ref[a1], tbl_ref[a2])

        for j in range(S):
            t8 = pl.multiple_of(meta_ref[base + 2 * S + j], 8)
            sy = meta_ref[base + 3 * S + j]
            ey = meta_ref[base + 4 * S + j]
            sl2 = p1_ref[:, pl.ds(t8, HWIN), :]
            ah = t8 + jax.lax.broadcasted_iota(jnp.int32, (1, HWIN, 1), 1)
            m2 = (ah >= sy) & (ah < ey)
            out_ref[0, rr, j] = jnp.max(jnp.where(m2, sl2, NEG), axis=1)


def _bin_meta(lo, hi, extent, win):
    n = hi - lo + 1
    i = jnp.arange(S, dtype=jnp.int32)
    start = lo[..., None] + (i * n[..., None]) // S
    end = lo[..., None] - ((-(i + 1) * n[..., None]) // S)
    width = end - start
    lvl = ((width >= 2).astype(jnp.int32) + (width >= 4).astype(jnp.int32)
           + (width >= 8).astype(jnp.int32))
    p = jnp.left_shift(jnp.int32(1), lvl)
    a1 = lvl * extent + start
    a2 = lvl * extent + end - p
    t8 = jnp.minimum((start >> 3) << 3, extent - win)
    return start, end, a1, a2, t8


def kernel(feature_map, rois):
    B, C, W, H = feature_map.shape
    R = rois.shape[1]
    fm_t = jnp.transpose(feature_map, (0, 2, 3, 1))
    boxes = rois.astype(jnp.int32)
    x1, y1, x2, y2 = (boxes[..., 0], boxes[..., 1],
                      boxes[..., 2], boxes[..., 3])
    _, _, xa1, xa2, _ = _bin_meta(x1, x2, W, HWIN)
    sy, ey, _, _, t8 = _bin_meta(y1, y2, H, HWIN)
    meta = jnp.concatenate([xa1, xa2, t8, sy, ey], axis=-1)
    meta = meta.reshape(-1)

    out = pl.pallas_call(
        _roi_kernel,
        grid=(B, R // RB),
        in_specs=[
            pl.BlockSpec(memory_space=pltpu.SMEM),
            pl.BlockSpec((1, W, H, C), lambda b, r: (b, 0, 0, 0)),
        ],
        out_specs=pl.BlockSpec((1, RB, S, S, C), lambda b, r: (b, r, 0, 0, 0)),
        out_shape=jax.ShapeDtypeStruct((B, R, S, S, C), jnp.float32),
        scratch_shapes=[
            pltpu.VMEM((S, H, C), jnp.float32),
            pltpu.VMEM((4 * W, H, C), jnp.float32),
        ],
        compiler_params=pltpu.CompilerParams(
            dimension_semantics=("parallel", "arbitrary"),
        ),
        name="roi_pool",
    )(meta, fm_t)
    return jnp.transpose(out, (0, 1, 4, 3, 2))


# speedup vs baseline: 22.7598x; 1.0111x over previous
import jax
import jax.numpy as jnp
import numpy as np
from jax.experimental import pallas as pl
from jax.experimental.pallas import tpu as pltpu

S = 7
RB = 32
HWIN = 24
MW = 5 * S

NEG = float(np.finfo(np.float32).min)


def _roi_kernel(meta_ref, fm_ref, out_ref, p1_ref, tbl_ref):
    b = pl.program_id(0)
    rblk = pl.program_id(1)
    _, W, H, C = fm_ref.shape
    R_total = out_ref.shape[1] * pl.num_programs(1)

    @pl.when(rblk == 0)
    def _build():
        tbl_ref[0 * W:0 * W + 64] = fm_ref[0]
        tbl_ref[1 * W:1 * W + 63] = jnp.maximum(tbl_ref[0:63], tbl_ref[1:64])
        tbl_ref[2 * W:2 * W + 61] = jnp.maximum(tbl_ref[W:W + 61],
                                                tbl_ref[W + 2:W + 63])
        tbl_ref[3 * W:3 * W + 57] = jnp.maximum(tbl_ref[2 * W:2 * W + 57],
                                                tbl_ref[2 * W + 4:2 * W + 61])

    for rr in range(RB):
        base = (b * R_total + rblk * RB + rr) * MW

        for i in range(S):
            a1 = meta_ref[base + i]
            a2 = meta_ref[base + S + i]
            p1_ref[i] = jnp.maximum(tbl_ref[a1], tbl_ref[a2])

        for j in range(S):
            t8 = pl.multiple_of(meta_ref[base + 2 * S + j], 8)
            sy = meta_ref[base + 3 * S + j]
            ey = meta_ref[base + 4 * S + j]
            sl2 = p1_ref[:, pl.ds(t8, HWIN), :]
            ah = t8 + jax.lax.broadcasted_iota(jnp.int32, (1, HWIN, 1), 1)
            m2 = (ah >= sy) & (ah < ey)
            out_ref[0, rr, j] = jnp.max(jnp.where(m2, sl2, NEG), axis=1)


def _bin_meta(lo, hi, extent, win):
    n = hi - lo + 1
    i = jnp.arange(S, dtype=jnp.int32)
    start = lo[..., None] + (i * n[..., None]) // S
    end = lo[..., None] - ((-(i + 1) * n[..., None]) // S)
    width = end - start
    lvl = ((width >= 2).astype(jnp.int32) + (width >= 4).astype(jnp.int32)
           + (width >= 8).astype(jnp.int32))
    p = jnp.left_shift(jnp.int32(1), lvl)
    a1 = lvl * extent + start
    a2 = lvl * extent + end - p
    t8 = jnp.minimum((start >> 3) << 3, extent - win)
    return start, end, a1, a2, t8


def kernel(feature_map, rois):
    B, C, W, H = feature_map.shape
    R = rois.shape[1]
    fm_t = jnp.transpose(feature_map, (0, 2, 3, 1))
    boxes = rois.astype(jnp.int32)
    x1, y1, x2, y2 = (boxes[..., 0], boxes[..., 1],
                      boxes[..., 2], boxes[..., 3])
    _, _, xa1, xa2, _ = _bin_meta(x1, x2, W, HWIN)
    sy, ey, _, _, t8 = _bin_meta(y1, y2, H, HWIN)
    meta = jnp.concatenate([xa1, xa2, t8, sy, ey], axis=-1)
    meta = meta.reshape(-1)

    out = pl.pallas_call(
        _roi_kernel,
        grid=(B, R // RB),
        in_specs=[
            pl.BlockSpec(memory_space=pltpu.SMEM),
            pl.BlockSpec((1, W, H, C), lambda b, r: (b, 0, 0, 0)),
        ],
        out_specs=pl.BlockSpec((1, RB, S, S, C), lambda b, r: (b, r, 0, 0, 0)),
        out_shape=jax.ShapeDtypeStruct((B, R, S, S, C), jnp.float32),
        scratch_shapes=[
            pltpu.VMEM((S, H, C), jnp.float32),
            pltpu.VMEM((4 * W, H, C), jnp.float32),
        ],
        compiler_params=pltpu.CompilerParams(
            dimension_semantics=("parallel", "arbitrary"),
        ),
        name="roi_pool",
    )(meta, fm_t)
    return jnp.transpose(out, (0, 1, 4, 3, 2))
